# baseline (speedup 1.0000x reference)
import functools

import jax
import jax.numpy as jnp
from jax import lax
from jax.experimental import pallas as pl
from jax.experimental.pallas import tpu as pltpu
from jax.experimental.pallas import tpu_sc as plsc

D0, D1, D2 = 100, 100, 100
B = 16384
NC, NS, L = 2, 16, 16
NW = NC * NS
BPW = B // NW
GROUPS = BPW // L

_mesh = plsc.VectorSubcoreMesh(core_axis_name="c", subcore_axis_name="s")


@functools.partial(
    pl.kernel,
    mesh=_mesh,
    out_type=jax.ShapeDtypeStruct((B,), jnp.float32),
    scratch_types=[
        pltpu.VMEM((3 * BPW,), jnp.int32),
        pltpu.VMEM((BPW,), jnp.int32),
        pltpu.VMEM((BPW,), jnp.float32),
        pltpu.SemaphoreType.DMA,
    ],
    compiler_params=pltpu.CompilerParams(needs_layout_passes=False),
)
def _clique_gather(x_hbm, w_hbm, out_hbm, xraw_v, idx_v, val_v, sem):
    wid = lax.axis_index("s") * NC + lax.axis_index("c")
    base = wid * BPW
    pltpu.sync_copy(x_hbm.at[pl.ds(base * 3, 3 * BPW)], xraw_v)
    lanes = lax.iota(jnp.int32, L)
    for g in range(GROUPS):
        pos = (lanes + g * L) * 3
        i0 = plsc.load_gather(xraw_v, [pos])
        i1 = plsc.load_gather(xraw_v, [pos + 1])
        i2 = plsc.load_gather(xraw_v, [pos + 2])
        idx_v[pl.ds(g * L, L)] = i0 * (D1 * D2) + i1 * D2 + i2
    pltpu.async_copy(w_hbm.at[idx_v], val_v, sem).wait()
    pltpu.sync_copy(val_v, out_hbm.at[pl.ds(base, BPW)])


def kernel(x, W):
    xf = x.reshape(-1).astype(jnp.int32)
    wf = W.reshape(-1)
    return _clique_gather(xf, wf).reshape(B, 1)
